# Initial kernel scaffold; baseline (speedup 1.0000x reference)
import jax
import jax.numpy as jnp
from jax.experimental import pallas as pl


def kernel(indices, table):
    raise NotImplementedError("write your pallas kernel here")


# speedup vs baseline: 5.3462x; 5.3462x over previous
import functools

import jax
import jax.numpy as jnp
from jax import lax
from jax.experimental import pallas as pl
from jax.experimental.pallas import tpu as pltpu
from jax.experimental.pallas import tpu_sc as plsc

_VOCAB = 101
_EMBED = 32
_BATCH = 16384
_HIST = 50

_NW = 32
_IDX_PER_STREAM = 128
_STREAMS_PER_CHUNK = 8
_CHUNK = _IDX_PER_STREAM * _STREAMS_PER_CHUNK


def _gather_kernel(idx_hbm, table_hbm, out_hbm, idx_v, rows_v, sem):
    nflat = _BATCH * _HIST
    b_per_w = nflat // _NW
    steps = b_per_w // _CHUNK
    wid = lax.axis_index("s") * 2 + lax.axis_index("c")
    idx_row0 = wid * (b_per_w // _IDX_PER_STREAM)

    def body(g, _):
        pltpu.sync_copy(
            idx_hbm.at[pl.ds(idx_row0 + g * _STREAMS_PER_CHUNK,
                             _STREAMS_PER_CHUNK)],
            idx_v)
        copies = [
            pltpu.async_copy(
                table_hbm.at[idx_v.at[j]],
                rows_v.at[pl.ds(j * _IDX_PER_STREAM, _IDX_PER_STREAM)],
                sem)
            for j in range(_STREAMS_PER_CHUNK)
        ]
        for c in copies:
            c.wait()
        pltpu.sync_copy(
            rows_v,
            out_hbm.at[pl.ds(wid * b_per_w + g * _CHUNK, _CHUNK)])
        return _

    lax.fori_loop(0, steps, body, None)


def kernel(indices, table):
    nflat = indices.shape[0] * indices.shape[1]
    idx2 = indices.reshape(nflat // _IDX_PER_STREAM, _IDX_PER_STREAM)
    mesh = plsc.VectorSubcoreMesh(core_axis_name="c", subcore_axis_name="s")
    call = pl.kernel(
        _gather_kernel,
        mesh=mesh,
        out_type=jax.ShapeDtypeStruct((nflat, _EMBED), jnp.float32),
        scratch_types=[
            pltpu.VMEM((_STREAMS_PER_CHUNK, _IDX_PER_STREAM), jnp.int32),
            pltpu.VMEM((_CHUNK, _EMBED), jnp.float32),
            pltpu.SemaphoreType.DMA,
        ],
        compiler_params=pltpu.CompilerParams(use_tc_tiling_on_sc=False),
    )
    out = call(idx2, table)
    return out.reshape(indices.shape[0], -1)


# speedup vs baseline: 5.3797x; 1.0063x over previous
import jax
import jax.numpy as jnp
from jax import lax
from jax.experimental import pallas as pl
from jax.experimental.pallas import tpu as pltpu
from jax.experimental.pallas import tpu_sc as plsc

_EMBED = 32
_BATCH = 16384
_HIST = 50

_NW = 32
_IDX_PER_STREAM = 128
_STREAMS_PER_CHUNK = 10
_CHUNK = _IDX_PER_STREAM * _STREAMS_PER_CHUNK
_NBUF = 2


def _gather_kernel(idx_hbm, table_hbm, out_hbm,
                   idx_v, rows_v, isem, gsem, osem):
    nflat = _BATCH * _HIST
    b_per_w = nflat // _NW
    steps = b_per_w // _CHUNK
    wid = lax.axis_index("s") * 2 + lax.axis_index("c")
    idx_row0 = wid * (b_per_w // _IDX_PER_STREAM)
    out_row0 = wid * b_per_w

    def idx_copy(c, b):
        return pltpu.make_async_copy(
            idx_hbm.at[pl.ds(idx_row0 + c * _STREAMS_PER_CHUNK,
                             _STREAMS_PER_CHUNK)],
            idx_v.at[b], isem.at[b])

    def gather_copy(b, j):
        return pltpu.make_async_copy(
            table_hbm.at[idx_v.at[b, j]],
            rows_v.at[b, pl.ds(j * _IDX_PER_STREAM, _IDX_PER_STREAM)],
            gsem.at[b])

    def writeout_copy(c, b):
        return pltpu.make_async_copy(
            rows_v.at[b],
            out_hbm.at[pl.ds(out_row0 + c * _CHUNK, _CHUNK)],
            osem.at[b])

    def fire_gathers(b):
        for j in range(_STREAMS_PER_CHUNK):
            gather_copy(b, j).start()

    def wait_gathers(b):
        for j in range(_STREAMS_PER_CHUNK):
            gather_copy(b, j).wait()

    for b in range(_NBUF):
        idx_copy(b, b).start()
        idx_copy(b, b).wait()
        fire_gathers(b)

    @pl.loop(0, steps - _NBUF, step=_NBUF)
    def _(g):
        for b in range(_NBUF):
            c = g + b
            wait_gathers(b)
            writeout_copy(c, b).start()
            idx_copy(c + _NBUF, b).start()
            writeout_copy(c, b).wait()
            idx_copy(c + _NBUF, b).wait()
            fire_gathers(b)

    for b in range(_NBUF):
        wait_gathers(b)
        writeout_copy(steps - _NBUF + b, b).start()
    for b in range(_NBUF):
        writeout_copy(steps - _NBUF + b, b).wait()


def kernel(indices, table):
    nflat = indices.shape[0] * indices.shape[1]
    idx2 = indices.reshape(nflat // _IDX_PER_STREAM, _IDX_PER_STREAM)
    mesh = plsc.VectorSubcoreMesh(core_axis_name="c", subcore_axis_name="s")
    call = pl.kernel(
        _gather_kernel,
        mesh=mesh,
        out_type=jax.ShapeDtypeStruct((nflat, _EMBED), jnp.float32),
        scratch_types=[
            pltpu.VMEM((_NBUF, _STREAMS_PER_CHUNK, _IDX_PER_STREAM),
                       jnp.int32),
            pltpu.VMEM((_NBUF, _CHUNK, _EMBED), jnp.float32),
            pltpu.SemaphoreType.DMA((_NBUF,)),
            pltpu.SemaphoreType.DMA((_NBUF,)),
            pltpu.SemaphoreType.DMA((_NBUF,)),
        ],
        compiler_params=pltpu.CompilerParams(use_tc_tiling_on_sc=False),
    )
    out = call(idx2, table)
    return out.reshape(indices.shape[0], -1)
